# Initial kernel scaffold; baseline (speedup 1.0000x reference)
import jax
import jax.numpy as jnp
from jax.experimental import pallas as pl


def kernel(x, edge_index, edge_attr, batch, W_ne, b_ne, W_ee, b_ee, W1, b1, W2, b2, t1, w_pool, W3, b3, W4, b4, t2, W_ai, b_ai, W_ao, b_ao):
    raise NotImplementedError("write your pallas kernel here")


# speedup vs baseline: 12.2643x; 12.2643x over previous
import functools

import jax
import jax.numpy as jnp
from jax import lax
from jax.experimental import pallas as pl
from jax.experimental.pallas import tpu as pltpu
from jax.experimental.pallas import tpu_sc as plsc

N = 10000
E = 320000
DF = 128
DE = 16
H = 16
G = 16
A = 8

NW = 32
PT = E // NW
CH = 80
NCH = PT // CH
NP = 10240
ROWS_PER_TILE = NP // 16
KSEL = N // 2


def _sc_conv_body(h_hbm, e_hbm, src_hbm, dst_hbm, par_hbm, mask_hbm, out_hbm,
                  srcbuf, dstbuf, maskbuf, pbuf, ebuf, hrows, valbuf, zbuf,
                  acc, sem_g, sem_e):
    c = lax.axis_index("c")
    s = lax.axis_index("s")
    wid = c * 16 + s

    pltpu.sync_copy(src_hbm.at[wid], srcbuf)
    pltpu.sync_copy(dst_hbm.at[wid], dstbuf)
    pltpu.sync_copy(mask_hbm, maskbuf)
    pltpu.sync_copy(par_hbm, pbuf)

    zero16 = jnp.zeros((16,), jnp.float32)

    def _zb(i, carry):
        zbuf[i, pl.ds(0, 16)] = zero16
        zbuf[i, pl.ds(16, 16)] = zero16
        return carry

    lax.fori_loop(0, CH, _zb, 0)

    def _zc(j, carry):
        pltpu.sync_copy(zbuf, acc.at[pl.ds(s * ROWS_PER_TILE + j * CH, CH)])
        return carry

    lax.fori_loop(0, ROWS_PER_TILE // CH, _zc, 0)

    trash16 = jnp.full((16,), N, jnp.int32)

    def _deff(ci, carry):
        def _q(q, carry2):
            su = srcbuf[ci, pl.ds(q * 16, 16)]
            du = dstbuf[ci, pl.ds(q * 16, 16)]
            ms = plsc.load_gather(maskbuf, [su])
            md = plsc.load_gather(maskbuf, [du])
            ok = (ms > 0) & (md > 0)
            dstbuf[ci, pl.ds(q * 16, 16)] = jnp.where(ok, du, trash16)
            return carry2

        lax.fori_loop(0, CH // 16, _q, 0)
        return carry

    lax.fori_loop(0, NCH, _deff, 0)

    tv = pbuf[0, :]
    bv = pbuf[1, :]

    plsc.subcore_barrier()

    def _chunk(ci, carry):
        base = wid * PT + ci * CH
        cp_e = pltpu.async_copy(e_hbm.at[pl.ds(base, CH)], ebuf, sem_e)
        cp_h = pltpu.async_copy(h_hbm.at[srcbuf.at[ci]], hrows, sem_g)
        cp_e.wait()
        cp_h.wait()

        def _edge(i, carry2):
            hv = hrows[i, :]
            ev = ebuf[i, :]
            mm = jnp.maximum(hv + ev, 0.0) + 1e-7
            exv = jnp.exp(tv * mm - bv)
            valbuf[i, pl.ds(0, 16)] = exv * mm
            valbuf[i, pl.ds(16, 16)] = exv
            return carry2

        lax.fori_loop(0, CH, _edge, 0)
        pltpu.sync_copy(valbuf, acc.at[dstbuf.at[ci]], add=True)
        return carry

    lax.fori_loop(0, NCH, _chunk, 0)

    plsc.subcore_barrier()
    pltpu.sync_copy(acc.at[pl.ds(s * ROWS_PER_TILE, ROWS_PER_TILE)],
                    out_hbm.at[c, pl.ds(s * ROWS_PER_TILE, ROWS_PER_TILE)])


def _build_sc_conv():
    mesh = plsc.VectorSubcoreMesh(core_axis_name="c", subcore_axis_name="s")
    return pl.kernel(
        _sc_conv_body,
        out_type=jax.ShapeDtypeStruct((2, NP, 32), jnp.float32),
        mesh=mesh,
        compiler_params=pltpu.CompilerParams(needs_layout_passes=False,
                                             use_tc_tiling_on_sc=False),
        scratch_types=[
            pltpu.VMEM((NCH, CH), jnp.int32),
            pltpu.VMEM((NCH, CH), jnp.int32),
            pltpu.VMEM((N,), jnp.int32),
            pltpu.VMEM((2, 16), jnp.float32),
            pltpu.VMEM((CH, 16), jnp.float32),
            pltpu.VMEM((CH, 16), jnp.float32),
            pltpu.VMEM((CH, 32), jnp.float32),
            pltpu.VMEM((CH, 32), jnp.float32),
            pltpu.VMEM_SHARED((NP, 32), jnp.float32),
            pltpu.SemaphoreType.DMA,
            pltpu.SemaphoreType.DMA,
        ],
    )


def _a_body(x_ref, w_ref, b_ref, h_ref, mx_ref):
    h = jnp.dot(x_ref[...], w_ref[...],
                preferred_element_type=jnp.float32) + b_ref[...]
    h_ref[...] = h
    mx_ref[...] = jnp.full((8, 128), jnp.max(h), jnp.float32)


def _proj(x, w, b, blocks):
    rows = x.shape[0] // blocks
    kdim = x.shape[1]
    odim = w.shape[1]
    return pl.pallas_call(
        _a_body,
        grid=(blocks,),
        in_specs=[
            pl.BlockSpec((rows, kdim), lambda i: (i, 0)),
            pl.BlockSpec((kdim, odim), lambda i: (0, 0)),
            pl.BlockSpec((1, odim), lambda i: (0, 0)),
        ],
        out_specs=[
            pl.BlockSpec((rows, odim), lambda i: (i, 0)),
            pl.BlockSpec((8, 128), lambda i: (i, 0)),
        ],
        out_shape=[
            jax.ShapeDtypeStruct((x.shape[0], odim), jnp.float32),
            jax.ShapeDtypeStruct((blocks * 8, 128), jnp.float32),
        ],
    )(x, w, b.reshape(1, odim))


def _b_body(h0_ref, a0_ref, a1_ref, w1_ref, b1_ref, w2_ref, b2_ref, wp_ref,
            hg_ref, s_ref, mx_ref):
    num = a0_ref[:, :16] + a1_ref[:, :16]
    den = a0_ref[:, 16:] + a1_ref[:, 16:]
    hh = h0_ref[...] + num / (den + 1e-16)
    z1 = jnp.maximum(jnp.dot(hh, w1_ref[...],
                             preferred_element_type=jnp.float32)
                     + b1_ref[...], 0.0)
    h1 = jnp.maximum(jnp.dot(z1, w2_ref[...],
                             preferred_element_type=jnp.float32)
                     + b2_ref[...], 0.0)
    wp = wp_ref[...]
    sv = jnp.dot(h1, wp, preferred_element_type=jnp.float32)
    nrm = jnp.sqrt(jnp.sum(wp * wp))
    g = jnp.tanh(sv / (nrm + 1e-16))
    hg = h1 * g
    hg_ref[...] = hg
    s_ref[...] = sv
    mx_ref[...] = jnp.full((8, 128), jnp.max(hg), jnp.float32)


def _node1(h0, a0, a1, w1, b1, w2, b2, wp, blocks):
    rows = N // blocks
    return pl.pallas_call(
        _b_body,
        grid=(blocks,),
        in_specs=[
            pl.BlockSpec((rows, H), lambda i: (i, 0)),
            pl.BlockSpec((rows, 32), lambda i: (i, 0)),
            pl.BlockSpec((rows, 32), lambda i: (i, 0)),
            pl.BlockSpec((H, 2 * H), lambda i: (0, 0)),
            pl.BlockSpec((1, 2 * H), lambda i: (0, 0)),
            pl.BlockSpec((2 * H, H), lambda i: (0, 0)),
            pl.BlockSpec((1, H), lambda i: (0, 0)),
            pl.BlockSpec((H, 1), lambda i: (0, 0)),
        ],
        out_specs=[
            pl.BlockSpec((rows, H), lambda i: (i, 0)),
            pl.BlockSpec((rows, 1), lambda i: (i, 0)),
            pl.BlockSpec((8, 128), lambda i: (i, 0)),
        ],
        out_shape=[
            jax.ShapeDtypeStruct((N, H), jnp.float32),
            jax.ShapeDtypeStruct((N, 1), jnp.float32),
            jax.ShapeDtypeStruct((blocks * 8, 128), jnp.float32),
        ],
    )(h0, a0, a1, w1, b1.reshape(1, 2 * H), w2, b2.reshape(1, H),
      wp.reshape(H, 1))


def _c_body(s_ref, batch_ref, maskf_ref, maski_ref, counts_ref):
    bits = lax.bitcast_convert_type(s_ref[...], jnp.int32)
    keys = jnp.where(bits >= 0, bits, bits ^ jnp.int32(0x7FFFFFFF))
    cntpos = jnp.sum((keys >= 0).astype(jnp.int32))
    t0 = jnp.where(cntpos >= KSEL, jnp.int32(0), jnp.int32(-2147483648))

    def _bit(i, t):
        cand = t | (jnp.int32(1) << (30 - i))
        cnt = jnp.sum((keys >= cand).astype(jnp.int32))
        return jnp.where(cnt >= KSEL, cand, t)

    t = lax.fori_loop(0, 31, _bit, t0)
    m = (keys >= t)
    mf = m.astype(jnp.float32)
    maskf_ref[...] = mf
    maski_ref[...] = m.astype(jnp.int32)
    gi = lax.broadcasted_iota(jnp.int32, (1, G), 1)
    onehot = (batch_ref[...] == gi).astype(jnp.float32)
    counts_ref[...] = jnp.sum(mf * onehot, axis=0, keepdims=True)


def _select(s, batch):
    return pl.pallas_call(
        _c_body,
        out_shape=[
            jax.ShapeDtypeStruct((N, 1), jnp.float32),
            jax.ShapeDtypeStruct((N, 1), jnp.int32),
            jax.ShapeDtypeStruct((1, G), jnp.float32),
        ],
    )(s, batch)


def _d_body(hg_ref, a0_ref, a1_ref, maskf_ref, batch_ref, counts_ref,
            w3_ref, b3_ref, w4_ref, b4_ref, wai_ref, bai_ref, wao_ref,
            bao_ref, out_ref, mol_scr):
    i = pl.program_id(0)

    @pl.when(i == 0)
    def _():
        mol_scr[...] = jnp.zeros_like(mol_scr)

    num = a0_ref[:, :16] + a1_ref[:, :16]
    den = a0_ref[:, 16:] + a1_ref[:, 16:]
    hh = hg_ref[...] + num / (den + 1e-16)
    z1 = jnp.maximum(jnp.dot(hh, w3_ref[...],
                             preferred_element_type=jnp.float32)
                     + b3_ref[...], 0.0)
    h2 = jnp.maximum(jnp.dot(z1, w4_ref[...],
                             preferred_element_type=jnp.float32)
                     + b4_ref[...], 0.0)
    gi = lax.broadcasted_iota(jnp.int32, (1, G), 1)
    invc = 1.0 / jnp.maximum(counts_ref[...], 1.0)
    p = (batch_ref[...] == gi).astype(jnp.float32) * maskf_ref[...] * invc
    mol_scr[...] += lax.dot_general(p, h2, (((0,), (0,)), ((), ())),
                                    preferred_element_type=jnp.float32)

    @pl.when(i == pl.num_programs(0) - 1)
    def _():
        aa = jnp.maximum(jnp.dot(mol_scr[...], wai_ref[...],
                                 preferred_element_type=jnp.float32)
                         + bai_ref[...], 0.0)
        out_ref[...] = jnp.tanh(jnp.dot(aa, wao_ref[...],
                                        preferred_element_type=jnp.float32)
                                + bao_ref[...])


def _node2(hg, a0, a1, maskf, batch, counts, w3, b3, w4, b4, wai, bai, wao,
           bao, blocks):
    rows = N // blocks
    return pl.pallas_call(
        _d_body,
        grid=(blocks,),
        in_specs=[
            pl.BlockSpec((rows, H), lambda i: (i, 0)),
            pl.BlockSpec((rows, 32), lambda i: (i, 0)),
            pl.BlockSpec((rows, 32), lambda i: (i, 0)),
            pl.BlockSpec((rows, 1), lambda i: (i, 0)),
            pl.BlockSpec((rows, 1), lambda i: (i, 0)),
            pl.BlockSpec((1, G), lambda i: (0, 0)),
            pl.BlockSpec((H, 2 * H), lambda i: (0, 0)),
            pl.BlockSpec((1, 2 * H), lambda i: (0, 0)),
            pl.BlockSpec((2 * H, H), lambda i: (0, 0)),
            pl.BlockSpec((1, H), lambda i: (0, 0)),
            pl.BlockSpec((H, 128), lambda i: (0, 0)),
            pl.BlockSpec((1, 128), lambda i: (0, 0)),
            pl.BlockSpec((128, A), lambda i: (0, 0)),
            pl.BlockSpec((1, A), lambda i: (0, 0)),
        ],
        out_specs=pl.BlockSpec((G, A), lambda i: (0, 0)),
        out_shape=jax.ShapeDtypeStruct((G, A), jnp.float32),
        scratch_shapes=[pltpu.VMEM((G, G), jnp.float32)],
    )(hg, a0, a1, maskf, batch, counts, w3, b3.reshape(1, 2 * H), w4,
      b4.reshape(1, H), wai, bai.reshape(1, 128), wao, bao.reshape(1, A))


def kernel(x, edge_index, edge_attr, batch, W_ne, b_ne, W_ee, b_ee, W1, b1,
           W2, b2, t1, w_pool, W3, b3, W4, b4, t2, W_ai, b_ai, W_ao, b_ao):
    src2 = edge_index[0].reshape(NW, NCH, CH)
    dst2 = edge_index[1].reshape(NW, NCH, CH)
    batch2 = batch.reshape(N, 1)

    h0, mxh = _proj(x, W_ne, b_ne, 10)
    e, mxe = _proj(edge_attr, W_ee, b_ee, 16)
    maxh = jnp.max(mxh)
    maxe = jnp.max(mxe)

    sc_conv = _build_sc_conv()

    b1g = jnp.maximum(t1, 0.0) * (jnp.maximum(maxh + maxe, 0.0) + 1e-7)
    par1 = jnp.stack([jnp.full((16,), t1, jnp.float32),
                      jnp.full((16,), 1.0, jnp.float32) * b1g])
    ones_mask = jnp.ones((N,), jnp.int32)
    acc1 = sc_conv(h0, e, src2, dst2, par1, ones_mask)

    hg, s, mxhg = _node1(h0, acc1[0, :N, :], acc1[1, :N, :],
                         W1, b1, W2, b2, w_pool, 10)

    maskf, maski, counts = _select(s, batch2)

    maxhg = jnp.max(mxhg)
    b2g = jnp.maximum(t2, 0.0) * (jnp.maximum(maxhg + maxe, 0.0) + 1e-7)
    par2 = jnp.stack([jnp.full((16,), t2, jnp.float32),
                      jnp.full((16,), 1.0, jnp.float32) * b2g])
    acc2 = sc_conv(hg, e, src2, dst2, par2, maski.reshape(N))

    return _node2(hg, acc2[0, :N, :], acc2[1, :N, :], maskf, batch2, counts,
                  W3, b3, W4, b4, W_ai, b_ai, W_ao, b_ao, 10)
